# Initial kernel scaffold; baseline (speedup 1.0000x reference)
import jax
import jax.numpy as jnp
from jax.experimental import pallas as pl


def kernel(X, Y, enc_weight, u_bias, dec_weight, dec_bias):
    raise NotImplementedError("write your pallas kernel here")


# speedup vs baseline: 1.3292x; 1.3292x over previous
import functools

import jax
import jax.numpy as jnp
from jax import lax
from jax.experimental import pallas as pl
from jax.experimental.pallas import tpu as pltpu
from jax.experimental.pallas import tpu_sc as plsc

NUM_MICROBES = 100000
NUM_METABOLITES = 128
LATENT_DIM = 64
B, S = 1024, 50
N = B * S

NEG_HALF_LOG_2PI = -0.9189385332046727
EPS = 1.1920928955078125e-07
LOG_EPS = -15.942385152878742
LOG_1MEPS = -1.1920930376163766e-07

_LG_POLY = (
    -0.5772070495946178,
    0.8222666127840628,
    -0.3986709390276555,
    0.25969254045876444,
    -0.17193044906438762,
    0.09475735591761751,
    -0.03507800606528588,
    0.006170092259299822,
)


_NW = 32
_BPW = N // _NW
_CH = 80
_NCH = _BPW // _CH


def _sc_gather(enc_weight, ub_flat, idx):
    mesh = plsc.VectorSubcoreMesh(core_axis_name="c", subcore_axis_name="s")

    @functools.partial(
        pl.kernel,
        mesh=mesh,
        compiler_params=pltpu.CompilerParams(use_tc_tiling_on_sc=False),
        out_type=[
            jax.ShapeDtypeStruct((N, LATENT_DIM), jnp.float32),
            jax.ShapeDtypeStruct((N,), jnp.float32),
        ],
        scratch_types=[
            pltpu.VMEM((_BPW,), jnp.int32),
            pltpu.VMEM((_BPW, LATENT_DIM), jnp.float32),
            pltpu.VMEM((_BPW,), jnp.float32),
            pltpu.SemaphoreType.DMA,
        ],
    )
    def gather_kernel(enc_hbm, ub_hbm, idx_hbm, z_hbm, ubg_hbm,
                      idx_v, rows_v, ubg_v, sem):
        wid = lax.axis_index("s") * 2 + lax.axis_index("c")
        base = wid * _BPW
        pltpu.sync_copy(idx_hbm.at[pl.ds(base, _BPW)], idx_v)
        copies = []
        for j in range(_NCH):
            sl = pl.ds(j * _CH, _CH)
            idx_sl = idx_v.at[sl]
            copies.append(pltpu.async_copy(enc_hbm.at[idx_sl], rows_v.at[sl, :], sem))
            copies.append(pltpu.async_copy(ub_hbm.at[idx_sl], ubg_v.at[sl], sem))
        for c in copies:
            c.wait()
        pltpu.sync_copy(rows_v, z_hbm.at[pl.ds(base, _BPW)])
        pltpu.sync_copy(ubg_v, ubg_hbm.at[pl.ds(base, _BPW)])

    return gather_kernel(enc_weight, ub_flat, idx)


_ROWS = 512
_NSTEP = N // _ROWS
_EROWS = NUM_MICROBES // _NSTEP


def _lgamma1p(t):
    acc = jnp.full_like(t, _LG_POLY[-1])
    for c in _LG_POLY[-2::-1]:
        acc = acc * t + c
    return acc * t


def _lgamma_big(x):
    acc = jnp.zeros_like(x)
    for _ in range(7):
        small = x < 8.0
        acc += jnp.where(small, jnp.log(x), 0.0)
        x = jnp.where(small, x + 1.0, x)
    xi = 1.0 / x
    lg = ((x - 0.5) * jnp.log(x) - x - NEG_HALF_LOG_2PI
          + xi * (1.0 / 12.0 - xi * xi * (1.0 / 360.0)))
    return lg - acc


def _tc_body(z_ref, ub_ref, y_ref, enc_ref, w_ref, b_ref, out_ref):
    i = pl.program_id(0)

    @pl.when(i == 0)
    def _():
        out_ref[0, 0] = 0.0

    z = z_ref[...] + ub_ref[...]
    logits_raw = (jnp.dot(z, w_ref[...], preferred_element_type=jnp.float32)
                  + b_ref[...])
    m = jnp.max(logits_raw, axis=1, keepdims=True)
    e = jnp.exp(logits_raw - m)
    lse = m + jnp.log(jnp.sum(e, axis=1, keepdims=True))
    lg = jnp.clip(logits_raw - lse, LOG_EPS, LOG_1MEPS)
    Y = y_ref[...]
    log_powers = jnp.sum(lg * Y, axis=1, keepdims=True)
    sum_lgy = jnp.sum(_lgamma1p(Y), axis=1, keepdims=True)
    ysum = jnp.sum(Y, axis=1, keepdims=True)
    lgn = _lgamma_big(ysum + 1.0)
    lp = lgn - sum_lgy + log_powers
    enc = enc_ref[...]
    out_ref[0, 0] += (jnp.sum(lp) * (1.0 / N) - 0.5 * jnp.sum(enc * enc))

    @pl.when(i == _NSTEP - 1)
    def _():
        w = w_ref[...]
        l_v = (-0.5 * jnp.sum(w * w)
               + (NUM_METABOLITES - 1) * LATENT_DIM * NEG_HALF_LOG_2PI)
        out_ref[0, 0] += l_v + NUM_MICROBES * LATENT_DIM * NEG_HALF_LOG_2PI


def _tc_compute(z, ub, y2, enc, wp, bp):
    return pl.pallas_call(
        _tc_body,
        grid=(_NSTEP,),
        in_specs=[
            pl.BlockSpec((_ROWS, LATENT_DIM), lambda i: (i, 0)),
            pl.BlockSpec((_ROWS, 1), lambda i: (i, 0)),
            pl.BlockSpec((_ROWS, NUM_METABOLITES), lambda i: (i, 0)),
            pl.BlockSpec((_EROWS, LATENT_DIM), lambda i: (i, 0)),
            pl.BlockSpec((LATENT_DIM, NUM_METABOLITES), lambda i: (0, 0)),
            pl.BlockSpec((1, NUM_METABOLITES), lambda i: (0, 0)),
        ],
        out_specs=pl.BlockSpec(memory_space=pltpu.SMEM),
        out_shape=jax.ShapeDtypeStruct((1, 1), jnp.float32),
    )(z, ub, y2, enc, wp, bp)


def kernel(X, Y, enc_weight, u_bias, dec_weight, dec_bias):
    idx = X.reshape(-1)
    z, ubg = _sc_gather(enc_weight, u_bias.reshape(-1), idx)
    wp = jnp.concatenate(
        [jnp.zeros((LATENT_DIM, 1), jnp.float32), dec_weight.T], axis=1)
    bp = jnp.concatenate(
        [jnp.zeros((1,), jnp.float32), dec_bias]).reshape(1, NUM_METABOLITES)
    out = _tc_compute(z, ubg.reshape(N, 1), Y.reshape(N, NUM_METABOLITES),
                      enc_weight, wp, bp)
    return out.reshape(())
